# Initial kernel scaffold; baseline (speedup 1.0000x reference)
import jax
import jax.numpy as jnp
from jax.experimental import pallas as pl


def kernel(images, predictions, anchor_boxes):
    raise NotImplementedError("write your pallas kernel here")


# speedup vs baseline: 9.3292x; 9.3292x over previous
import functools
import jax
import jax.numpy as jnp
from jax import lax
from jax.experimental import pallas as pl
from jax.experimental.pallas import tpu as pltpu
from jax.experimental.pallas import tpu_sc as plsc

_B = 8
_N = 20000
_NP = 20480
_CH = 2560
_NSTEP = _NP // _CH
_NCLS = 80
_MAXDET = 100
_NEG = -1e9
_OFF = 4.0 * 512.0

_NSC = 4
_CHUNK = _NP // _NSC
_NCK = _CHUNK // 16


def _decode_kernel(p_ref, anc_ref, nms_ref, dec_ref):
    mx = p_ref[4]
    cidf = jnp.zeros_like(mx)
    for c in range(1, _NCLS):
        v = p_ref[4 + c]
        upd = v > mx
        mx = jnp.where(upd, v, mx)
        cidf = jnp.where(upd, jnp.float32(c), cidf)
    conf = 1.0 / (1.0 + jnp.exp(-mx))
    score = jnp.where(conf > jnp.float32(0.05), conf, _NEG)

    acx = jnp.broadcast_to(anc_ref[0], mx.shape)
    acy = jnp.broadcast_to(anc_ref[1], mx.shape)
    aw = jnp.broadcast_to(anc_ref[2], mx.shape)
    ah = jnp.broadcast_to(anc_ref[3], mx.shape)

    cx = (p_ref[0] * jnp.float32(0.1)) * aw + acx
    cy = (p_ref[1] * jnp.float32(0.1)) * ah + acy
    w = jnp.exp(p_ref[2] * jnp.float32(0.2)) * aw
    h = jnp.exp(p_ref[3] * jnp.float32(0.2)) * ah

    off = cidf * _OFF
    x1o = (cx - w * 0.5) + off
    y1o = (cy - h * 0.5) + off
    x2o = (cx + w * 0.5) + off
    y2o = (cy + h * 0.5) + off
    area = jnp.maximum(x2o - x1o, 0.0) * jnp.maximum(y2o - y1o, 0.0)

    nms_ref[0] = score
    nms_ref[1] = x1o
    nms_ref[2] = y1o
    nms_ref[3] = x2o
    nms_ref[4] = y2o
    nms_ref[5] = area
    dec_ref[0] = cx
    dec_ref[1] = cy
    dec_ref[2] = w
    dec_ref[3] = h
    dec_ref[4] = cidf


def _decode_call(p_t, anc):
    return pl.pallas_call(
        _decode_kernel,
        grid=(_NSTEP,),
        in_specs=[
            pl.BlockSpec((4 + _NCLS, _B, _CH), lambda j: (0, 0, j)),
            pl.BlockSpec((4, 1, _CH), lambda j: (0, 0, j)),
        ],
        out_specs=[
            pl.BlockSpec((6, _B, _CH), lambda j: (0, 0, j)),
            pl.BlockSpec((5, _B, _CH), lambda j: (0, 0, j)),
        ],
        out_shape=[
            jax.ShapeDtypeStruct((6, _B, _NP), jnp.float32),
            jax.ShapeDtypeStruct((5, _B, _NP), jnp.float32),
        ],
    )(p_t, anc)


def _sc_nms_body(nms_hbm, oidx_hbm, oscr_hbm,
                 s_v, x1_v, y1_v, x2_v, y2_v, ab_v,
                 rec_v, grp_v, oi_v, os_v, sh):
    cid = lax.axis_index("c")
    sid = lax.axis_index("s")
    batch = cid * 4 + sid // _NSC
    chunk = sid % _NSC
    base = pl.multiple_of(chunk * _CHUNK, _CHUNK)

    pltpu.sync_copy(nms_hbm.at[0, batch, pl.ds(base, _CHUNK)], s_v)
    pltpu.sync_copy(nms_hbm.at[1, batch, pl.ds(base, _CHUNK)], x1_v)
    pltpu.sync_copy(nms_hbm.at[2, batch, pl.ds(base, _CHUNK)], y1_v)
    pltpu.sync_copy(nms_hbm.at[3, batch, pl.ds(base, _CHUNK)], x2_v)
    pltpu.sync_copy(nms_hbm.at[4, batch, pl.ds(base, _CHUNK)], y2_v)
    pltpu.sync_copy(nms_hbm.at[5, batch, pl.ds(base, _CHUNK)], ab_v)

    iota16 = lax.broadcasted_iota(jnp.int32, (16,), 0)
    g0 = (sid // _NSC) * _NSC

    def amx(k, c):
        m16, ki = c
        v = s_v[pl.ds(pl.multiple_of(k * 16, 16), 16)]
        upd = v > m16
        return (jnp.where(upd, v, m16), jnp.where(upd, k, ki))

    carry0 = lax.fori_loop(
        0, _NCK, amx,
        (jnp.full((16,), -3e38, jnp.float32),
         jnp.zeros((16,), jnp.int32)), unroll=8)

    def step(i, carry):
        m16, ki = carry
        mm = m16
        gg = base + ki * 16 + iota16
        for shx in (1, 2, 4, 8):
            pidx = jnp.bitwise_xor(iota16, shx)
            pm = mm.at[pidx].get(mode="promise_in_bounds")
            pg = gg.at[pidx].get(mode="promise_in_bounds")
            upd = (pm > mm) | ((pm == mm) & (pg < gg))
            mm = jnp.where(upd, pm, mm)
            gg = jnp.where(upd, pg, gg)
        bidx = gg[0]

        loc = bidx - base
        lane = lax.rem(loc, 16)
        koff = pl.multiple_of(loc - lane, 16)
        lanev = iota16 * 0 + lane
        c1 = x1_v[pl.ds(koff, 16)].at[lanev].get(mode="promise_in_bounds")
        c2 = y1_v[pl.ds(koff, 16)].at[lanev].get(mode="promise_in_bounds")
        c3 = x2_v[pl.ds(koff, 16)].at[lanev].get(mode="promise_in_bounds")
        c4 = y2_v[pl.ds(koff, 16)].at[lanev].get(mode="promise_in_bounds")

        rec = jnp.where(iota16 == 0, mm,
              jnp.where(iota16 == 1, gg.astype(jnp.float32),
              jnp.where(iota16 == 2, c1,
              jnp.where(iota16 == 3, c2,
              jnp.where(iota16 == 4, c3,
              jnp.where(iota16 == 5, c4, jnp.float32(0.0)))))))
        rec_v[pl.ds(0, 16)] = rec
        pltpu.sync_copy(rec_v, sh.at[sid])
        plsc.subcore_barrier()

        pltpu.sync_copy(sh.at[pl.ds(g0, _NSC)], grp_v)
        r0 = grp_v[0, pl.ds(0, 16)]
        bs, bif = r0[0], r0[1]
        bx1, by1, bx2, by2 = r0[2], r0[3], r0[4], r0[5]
        for r in range(1, _NSC):
            rr = grp_v[r, pl.ds(0, 16)]
            rs, ri = rr[0], rr[1]
            bet = (rs > bs) | ((rs == bs) & (ri < bif))
            bs = jnp.where(bet, rs, bs)
            bif = jnp.where(bet, ri, bif)
            bx1 = jnp.where(bet, rr[2], bx1)
            by1 = jnp.where(bet, rr[3], by1)
            bx2 = jnp.where(bet, rr[4], bx2)
            by2 = jnp.where(bet, rr[5], by2)
        bii = bif.astype(jnp.int32)

        @pl.when(chunk == 0)
        def _record():
            for r in range(_MAXDET // 16 + 1):
                dsr = pl.ds(r * 16, 16)
                hit = (r * 16 + iota16) == i
                oi_v[dsr] = jnp.where(hit, bii, oi_v[dsr])
                os_v[dsr] = jnp.where(hit, bs, os_v[dsr])

        garea = (jnp.maximum(bx2 - bx1, 0.0) *
                 jnp.maximum(by2 - by1, 0.0) + jnp.float32(1e-8))

        def supk(k, c):
            nm16, nki = c
            dsk = pl.ds(pl.multiple_of(k * 16, 16), 16)
            x1 = x1_v[dsk]
            y1 = y1_v[dsk]
            x2 = x2_v[dsk]
            y2 = y2_v[dsk]
            inter = (jnp.maximum(jnp.minimum(x2, bx2) - jnp.maximum(x1, bx1), 0.0)
                     * jnp.maximum(jnp.minimum(y2, by2) - jnp.maximum(y1, by1), 0.0))
            rhs = ab_v[dsk] + garea
            gi = base + k * 16 + iota16
            sup = (3.0 * inter > rhs) | (gi == bii)
            ns = jnp.where(sup, jnp.float32(_NEG), s_v[dsk])
            s_v[dsk] = ns
            upd = ns > nm16
            return (jnp.where(upd, ns, nm16), jnp.where(upd, k, nki))

        ncarry = lax.fori_loop(
            0, _NCK, supk,
            (jnp.full((16,), -3e38, jnp.float32),
             jnp.zeros((16,), jnp.int32)), unroll=8)
        plsc.subcore_barrier()
        return ncarry

    lax.fori_loop(0, _MAXDET, step, carry0)

    @pl.when(chunk == 0)
    def _writeout():
        pltpu.sync_copy(oi_v, oidx_hbm.at[batch])
        pltpu.sync_copy(os_v, oscr_hbm.at[batch])


def _sc_nms_call(nms_in):
    mesh = plsc.VectorSubcoreMesh(core_axis_name="c", subcore_axis_name="s")
    f = pl.kernel(
        _sc_nms_body,
        out_type=[
            jax.ShapeDtypeStruct((_B, 128), jnp.int32),
            jax.ShapeDtypeStruct((_B, 128), jnp.float32),
        ],
        mesh=mesh,
        scratch_types=[
            pltpu.VMEM((_CHUNK,), jnp.float32),
            pltpu.VMEM((_CHUNK,), jnp.float32),
            pltpu.VMEM((_CHUNK,), jnp.float32),
            pltpu.VMEM((_CHUNK,), jnp.float32),
            pltpu.VMEM((_CHUNK,), jnp.float32),
            pltpu.VMEM((_CHUNK,), jnp.float32),
            pltpu.VMEM((128,), jnp.float32),
            pltpu.VMEM((_NSC, 128), jnp.float32),
            pltpu.VMEM((128,), jnp.int32),
            pltpu.VMEM((128,), jnp.float32),
            pltpu.VMEM_SHARED((16, 128), jnp.float32),
        ],
    )
    return f(nms_in)


def kernel(images, predictions, anchor_boxes):
    del images
    p = jnp.pad(predictions, ((0, 0), (0, _NP - _N), (0, 0)),
                constant_values=-1e9)
    p_t = jnp.transpose(p, (2, 0, 1))
    anc = jnp.pad(anchor_boxes, ((0, _NP - _N), (0, 0)))
    anc_t = jnp.transpose(anc, (1, 0)).reshape(4, 1, _NP)

    nms_in, dec = _decode_call(p_t, anc_t)
    bi, mv = _sc_nms_call(nms_in)

    idx = bi[:, :_MAXDET]
    conf = mv[:, :_MAXDET]
    valid = conf > jnp.float32(-1e8)
    cols = [jnp.take_along_axis(dec[k], idx, axis=1) for k in range(5)]
    out = jnp.stack(cols + [conf], axis=-1)
    return jnp.where(valid[..., None], out, jnp.float32(-1.0))


# speedup vs baseline: 20.5636x; 2.2042x over previous
import jax
import jax.numpy as jnp
from jax.experimental import pallas as pl
from jax.experimental.pallas import tpu as pltpu

_B = 8
_N = 20000
_NP = 20480
_CH = 2560
_NSTEP = _NP // _CH
_NCLS = 80
_MAXDET = 100
_NEG = -1e9
_OFF = 4.0 * 512.0


def _fused_kernel(p_ref, anc_ref, dec_ref, bi_ref, mv_ref,
                  s_s, x1_s, y1_s, x2_s, y2_s, ab_s):
    j = pl.program_id(0)

    mx = p_ref[4]
    cidf = jnp.zeros_like(mx)
    for c in range(1, _NCLS):
        v = p_ref[4 + c]
        upd = v > mx
        mx = jnp.where(upd, v, mx)
        cidf = jnp.where(upd, jnp.float32(c), cidf)
    conf = 1.0 / (1.0 + jnp.exp(-mx))
    score = jnp.where(conf > jnp.float32(0.05), conf, _NEG)

    acx = jnp.broadcast_to(anc_ref[0], mx.shape)
    acy = jnp.broadcast_to(anc_ref[1], mx.shape)
    aw = jnp.broadcast_to(anc_ref[2], mx.shape)
    ah = jnp.broadcast_to(anc_ref[3], mx.shape)

    cx = (p_ref[0] * jnp.float32(0.1)) * aw + acx
    cy = (p_ref[1] * jnp.float32(0.1)) * ah + acy
    w = jnp.exp(p_ref[2] * jnp.float32(0.2)) * aw
    h = jnp.exp(p_ref[3] * jnp.float32(0.2)) * ah

    off = cidf * _OFF
    x1o = (cx - w * 0.5) + off
    y1o = (cy - h * 0.5) + off
    x2o = (cx + w * 0.5) + off
    y2o = (cy + h * 0.5) + off
    area = jnp.maximum(x2o - x1o, 0.0) * jnp.maximum(y2o - y1o, 0.0)

    lanes = pl.ds(j * _CH, _CH)
    s_s[:, lanes] = score
    x1_s[:, lanes] = x1o
    y1_s[:, lanes] = y1o
    x2_s[:, lanes] = x2o
    y2_s[:, lanes] = y2o
    ab_s[:, lanes] = area
    dec_ref[0] = cx
    dec_ref[1] = cy
    dec_ref[2] = w
    dec_ref[3] = h
    dec_ref[4] = cidf

    @pl.when(j == _NSTEP - 1)
    def _nms():
        def body(i, carry):
            bia, mva = carry
            s = s_s[...]
            m = jnp.max(s, axis=1, keepdims=True)
            idxs = jax.lax.broadcasted_iota(jnp.int32, (_B, _NP), 1)
            bi = jnp.min(jnp.where(s == m, idxs, jnp.int32(2**30)),
                         axis=1, keepdims=True)
            mask = idxs == bi

            x1 = x1_s[...]
            y1 = y1_s[...]
            x2 = x2_s[...]
            y2 = y2_s[...]
            gx1 = jnp.max(jnp.where(mask, x1, _NEG), axis=1, keepdims=True)
            gy1 = jnp.max(jnp.where(mask, y1, _NEG), axis=1, keepdims=True)
            gx2 = jnp.max(jnp.where(mask, x2, _NEG), axis=1, keepdims=True)
            gy2 = jnp.max(jnp.where(mask, y2, _NEG), axis=1, keepdims=True)
            garea = jnp.maximum(gx2 - gx1, 0.0) * jnp.maximum(gy2 - gy1, 0.0)

            ix1 = jnp.maximum(x1, gx1)
            iy1 = jnp.maximum(y1, gy1)
            ix2 = jnp.minimum(x2, gx2)
            iy2 = jnp.minimum(y2, gy2)
            inter = jnp.maximum(ix2 - ix1, 0.0) * jnp.maximum(iy2 - iy1, 0.0)
            rhs = ab_s[...] + (garea + jnp.float32(1e-8))
            sup = (3.0 * inter > rhs) | mask
            s_s[...] = jnp.where(sup, _NEG, s)

            l128 = jax.lax.broadcasted_iota(jnp.int32, (_B, 128), 1)
            hit = l128 == i
            bia = jnp.where(hit, jnp.broadcast_to(bi, (_B, 128)), bia)
            mva = jnp.where(hit, jnp.broadcast_to(m, (_B, 128)), mva)
            return bia, mva

        init = (jnp.zeros((_B, 128), jnp.int32),
                jnp.zeros((_B, 128), jnp.float32))
        bia, mva = jax.lax.fori_loop(0, _MAXDET, body, init)
        bi_ref[...] = bia
        mv_ref[...] = mva


def _fused_call(p_t, anc):
    return pl.pallas_call(
        _fused_kernel,
        grid=(_NSTEP,),
        in_specs=[
            pl.BlockSpec((4 + _NCLS, _B, _CH), lambda j: (0, 0, j)),
            pl.BlockSpec((4, 1, _CH), lambda j: (0, 0, j)),
        ],
        out_specs=[
            pl.BlockSpec((5, _B, _CH), lambda j: (0, 0, j)),
            pl.BlockSpec((_B, 128), lambda j: (0, 0)),
            pl.BlockSpec((_B, 128), lambda j: (0, 0)),
        ],
        out_shape=[
            jax.ShapeDtypeStruct((5, _B, _NP), jnp.float32),
            jax.ShapeDtypeStruct((_B, 128), jnp.int32),
            jax.ShapeDtypeStruct((_B, 128), jnp.float32),
        ],
        scratch_shapes=[pltpu.VMEM((_B, _NP), jnp.float32)
                        for _ in range(6)],
    )(p_t, anc)


def kernel(images, predictions, anchor_boxes):
    del images
    p = jnp.pad(predictions, ((0, 0), (0, _NP - _N), (0, 0)),
                constant_values=-1e9)
    p_t = jnp.transpose(p, (2, 0, 1))
    anc = jnp.pad(anchor_boxes, ((0, _NP - _N), (0, 0)))
    anc_t = jnp.transpose(anc, (1, 0)).reshape(4, 1, _NP)

    dec, bi, mv = _fused_call(p_t, anc_t)

    idx = bi[:, :_MAXDET]
    conf = mv[:, :_MAXDET]
    valid = conf > jnp.float32(-1e8)
    cols = [jnp.take_along_axis(dec[k], idx, axis=1) for k in range(5)]
    out = jnp.stack(cols + [conf], axis=-1)
    return jnp.where(valid[..., None], out, jnp.float32(-1.0))
